# Initial kernel scaffold; baseline (speedup 1.0000x reference)
import jax
import jax.numpy as jnp
from jax.experimental import pallas as pl


def kernel(X, W, Wdiag, node_idx, edge_idx):
    raise NotImplementedError("write your pallas kernel here")


# speedup vs baseline: 2.8657x; 2.8657x over previous
import functools

import jax
import jax.numpy as jnp
from jax import lax
from jax.experimental import pallas as pl
from jax.experimental.pallas import tpu as pltpu
from jax.experimental.pallas import tpu_sc as plsc

N_NODES = 10000
N_HEDGES = 5000
NNZ = 320000
D = 128
DA = 144

NC = 2
NS = 16
NW = NC * NS

CHUNK = 128
NNZ_PAD = 327680
N_CHUNKS = NNZ_PAD // CHUNK
CPW = N_CHUNKS // NW

N_SRC = 10240
E_PAD = 5120
V_PAD = 10240


def _make_sc_aggregate(t_pad):
    rpt = t_pad // NS

    mesh = plsc.VectorSubcoreMesh(
        core_axis_name="c", subcore_axis_name="s", num_cores=NC, num_subcores=NS
    )

    @functools.partial(
        pl.kernel,
        out_type=jax.ShapeDtypeStruct((NC * t_pad, DA), jnp.float32),
        mesh=mesh,
        compiler_params=pltpu.CompilerParams(use_tc_tiling_on_sc=False),
        scratch_types=[
            pltpu.VMEM((CPW, CHUNK), jnp.int32),
            pltpu.VMEM((CPW, CHUNK), jnp.int32),
            pltpu.VMEM((CHUNK, DA), jnp.float32),
            pltpu.SemaphoreType.DMA,
            pltpu.VMEM_SHARED((t_pad, DA), jnp.float32),
        ],
    )
    def agg(src_hbm, gidx_hbm, sidx_hbm, zeros_hbm, out_hbm,
            gidx_v, sidx_v, rows_v, sem, acc):
        c = lax.axis_index("c")
        s = lax.axis_index("s")
        wid = c * NS + s

        for off in range(0, rpt, 640):
            sz = min(640, rpt - off)
            pltpu.sync_copy(zeros_hbm.at[pl.ds(0, sz)],
                            acc.at[pl.ds(s * rpt + off, sz)])
        plsc.subcore_barrier()

        base = wid * CPW
        pltpu.sync_copy(gidx_hbm.at[pl.ds(base, CPW)], gidx_v)
        pltpu.sync_copy(sidx_hbm.at[pl.ds(base, CPW)], sidx_v)

        def chunk_body(j, carry):
            pltpu.async_copy(src_hbm.at[gidx_v.at[j]], rows_v, sem).wait()
            pltpu.sync_copy(rows_v, acc.at[sidx_v.at[j]], add=True)
            return carry

        lax.fori_loop(0, CPW, chunk_body, 0)
        plsc.subcore_barrier()

        pltpu.sync_copy(acc.at[pl.ds(s * rpt, rpt)],
                        out_hbm.at[pl.ds(c * t_pad + s * rpt, rpt)])

    return agg


_sc_agg_edges = _make_sc_aggregate(E_PAD)
_sc_agg_nodes = _make_sc_aggregate(V_PAD)


BM = 1024


def _mm_body(x_ref, w_ref, o_ref):
    xp = lax.dot_general(x_ref[...], w_ref[...],
                         (((1,), (1,)), ((), ())),
                         preferred_element_type=jnp.float32)
    padded = jnp.concatenate(
        [xp, jnp.zeros((xp.shape[0], DA - D), jnp.float32)], axis=1)
    col = lax.broadcasted_iota(jnp.int32, padded.shape, 1)
    o_ref[...] = jnp.where(col == D, 1.0, padded)


def _edge_combine_body(a_ref, b_ref, wd_ref, o_ref):
    ssum = a_ref[...] + b_ref[...]
    cnt = ssum[:, D:D + 1]
    scale = wd_ref[...] / jnp.maximum(cnt, 1.0)
    col = lax.broadcasted_iota(jnp.int32, ssum.shape, 1)
    o_ref[...] = jnp.where(col < D, ssum * scale,
                           jnp.where(col == D, 1.0, 0.0))


def _node_combine_body(a_ref, b_ref, o_ref):
    ssum = a_ref[...] + b_ref[...]
    cnt = ssum[:, D:D + 1]
    o_ref[...] = ssum[:, :D] / jnp.maximum(cnt, 1.0)


def kernel(X, W, Wdiag, node_idx, edge_idx):
    ni = node_idx.astype(jnp.int32)
    ei = edge_idx.astype(jnp.int32)
    npad = NNZ_PAD - NNZ
    pad_n = jnp.full((npad,), N_NODES, jnp.int32)
    pad_e = jnp.full((npad,), N_HEDGES, jnp.int32)

    g1 = jnp.concatenate([ni, pad_n]).reshape(N_CHUNKS, CHUNK)
    s1 = jnp.concatenate([ei, pad_e]).reshape(N_CHUNKS, CHUNK)
    g2 = jnp.concatenate([ei, pad_e]).reshape(N_CHUNKS, CHUNK)
    s2 = jnp.concatenate([ni, pad_n]).reshape(N_CHUNKS, CHUNK)

    x_pad = jnp.zeros((N_SRC, D), jnp.float32).at[:N_NODES].set(X)
    wd_pad = jnp.zeros((E_PAD, 1), jnp.float32).at[:N_HEDGES, 0].set(Wdiag)
    zeros_blk = jnp.zeros((640, DA), jnp.float32)

    xp_aug = pl.pallas_call(
        _mm_body,
        grid=(N_SRC // BM,),
        in_specs=[
            pl.BlockSpec((BM, D), lambda i: (i, 0)),
            pl.BlockSpec((D, D), lambda i: (0, 0)),
        ],
        out_specs=pl.BlockSpec((BM, DA), lambda i: (i, 0)),
        out_shape=jax.ShapeDtypeStruct((N_SRC, DA), jnp.float32),
    )(x_pad, W)

    pe = _sc_agg_edges(xp_aug, g1, s1, zeros_blk)
    pe = pe.reshape(NC, E_PAD, DA)

    eb = 640
    xe_aug = pl.pallas_call(
        _edge_combine_body,
        grid=(E_PAD // eb,),
        in_specs=[
            pl.BlockSpec((eb, DA), lambda i: (i, 0)),
            pl.BlockSpec((eb, DA), lambda i: (i, 0)),
            pl.BlockSpec((eb, 1), lambda i: (i, 0)),
        ],
        out_specs=pl.BlockSpec((eb, DA), lambda i: (i, 0)),
        out_shape=jax.ShapeDtypeStruct((E_PAD, DA), jnp.float32),
    )(pe[0], pe[1], wd_pad)

    pv = _sc_agg_nodes(xe_aug, g2, s2, zeros_blk)
    pv = pv.reshape(NC, V_PAD, DA)

    xv = pl.pallas_call(
        _node_combine_body,
        grid=(V_PAD // BM,),
        in_specs=[
            pl.BlockSpec((BM, DA), lambda i: (i, 0)),
            pl.BlockSpec((BM, DA), lambda i: (i, 0)),
        ],
        out_specs=pl.BlockSpec((BM, D), lambda i: (i, 0)),
        out_shape=jax.ShapeDtypeStruct((V_PAD, D), jnp.float32),
    )(pv[0], pv[1])

    return xv[:N_NODES]
